# baseline (speedup 1.0000x reference)
import functools

import jax
import jax.numpy as jnp
from jax import lax
from jax.experimental import pallas as pl
from jax.experimental.pallas import tpu as pltpu
from jax.experimental.pallas import tpu_sc as plsc

N_NODES = 10000
N_EDGES = 160000
NPAD = 10240
DUMMY = N_NODES
NTILE = 16
BEDGE = 128
NB = 80
EPAD = NTILE * NB * BEDGE
ROWS_PER_TILE = NPAD // NTILE
BM = 256


def _make_sc_agg(nchk, with_deg):
  mesh = plsc.VectorSubcoreMesh(core_axis_name="c", subcore_axis_name="s")

  out_type = tuple(
      jax.ShapeDtypeStruct((NPAD, 128), jnp.float32)
      for _ in range(nchk + (1 if with_deg else 0)))
  scratch = [
      pltpu.VMEM((NB, BEDGE), jnp.int32),
      pltpu.VMEM((BEDGE, 128), jnp.float32),
      pltpu.VMEM((BEDGE, 128), jnp.float32),
      pltpu.VMEM((1, BEDGE), jnp.int32),
      pltpu.VMEM((1, BEDGE), jnp.int32),
      pltpu.VMEM((1, BEDGE), jnp.int32),
      pltpu.VMEM((1, BEDGE), jnp.int32),
      pltpu.VMEM_SHARED((NPAD, 128), jnp.float32),
      pltpu.SemaphoreType.DMA,
      pltpu.SemaphoreType.DMA,
  ]

  @functools.partial(pl.kernel, mesh=mesh, out_type=out_type,
                     scratch_types=scratch)
  def k(*refs):
    vals = refs[:nchk]
    pos = nchk
    packed, zeros_h = refs[pos], refs[pos + 1]
    pos += 2
    if with_deg:
      ones_h = refs[pos]
      pos += 1
    outs = refs[pos:pos + nchk]
    pos += nchk
    if with_deg:
      dout = refs[pos]
      pos += 1
    pk_v, rows0, rows1, sb0, sb1, db0, db1, acc, sem0, sem1 = refs[pos:]

    c = lax.axis_index("c")
    s = lax.axis_index("s")
    row0 = s * ROWS_PER_TILE

    pltpu.sync_copy(packed.at[s], pk_v)

    def zero_acc():
      for kk in range(ROWS_PER_TILE // 128):
        pltpu.sync_copy(zeros_h, acc.at[pl.ds(row0 + kk * 128, 128)])

    def flush(out):
      pltpu.sync_copy(acc.at[pl.ds(row0, ROWS_PER_TILE)],
                      out.at[pl.ds(row0, ROWS_PER_TILE)])

    def unpack(j, sb, db, need_src=True):
      for kk in range(BEDGE // 16):
        v = pk_v[j, pl.ds(kk * 16, 16)]
        if need_src:
          sb[0, pl.ds(kk * 16, 16)] = v & 0xFFFF
        db[0, pl.ds(kk * 16, 16)] = lax.shift_right_logical(v, 16)

    @pl.when(c == 0)
    def _():
      for ck in range(nchk):
        vck = vals[ck]
        zero_acc()
        plsc.subcore_barrier()

        unpack(0, sb0, db0)
        pltpu.async_copy(vck.at[sb0.at[0]], rows0, sem0)
        unpack(1, sb1, db1)

        def pair(i, carry):
          j = 2 * i
          pltpu.make_async_copy(vck.at[sb0.at[0]], rows0, sem0).wait()
          pltpu.async_copy(vck.at[sb1.at[0]], rows1, sem1)
          pltpu.sync_copy(rows0, acc.at[db0.at[0]], add=True)
          unpack(j + 2, sb0, db0)
          pltpu.make_async_copy(vck.at[sb1.at[0]], rows1, sem1).wait()
          pltpu.async_copy(vck.at[sb0.at[0]], rows0, sem0)
          pltpu.sync_copy(rows1, acc.at[db1.at[0]], add=True)
          unpack(j + 3, sb1, db1)
          return carry

        lax.fori_loop(0, NB // 2 - 1, pair, 0)
        pltpu.make_async_copy(vck.at[sb0.at[0]], rows0, sem0).wait()
        pltpu.async_copy(vck.at[sb1.at[0]], rows1, sem1)
        pltpu.sync_copy(rows0, acc.at[db0.at[0]], add=True)
        pltpu.make_async_copy(vck.at[sb1.at[0]], rows1, sem1).wait()
        pltpu.sync_copy(rows1, acc.at[db1.at[0]], add=True)
        plsc.subcore_barrier()

        flush(outs[ck])
        plsc.subcore_barrier()

    if with_deg:
      @pl.when(c == 1)
      def _():
        pltpu.sync_copy(ones_h, rows0)
        zero_acc()
        plsc.subcore_barrier()

        def dbatch(j, carry):
          unpack(j, sb0, db0, need_src=False)
          pltpu.sync_copy(rows0, acc.at[db0.at[0]], add=True)
          return carry

        lax.fori_loop(0, NB, dbatch, 0)
        plsc.subcore_barrier()
        flush(dout)

  return k


def _deg_inv(pd_ref):
  return 1.0 / jnp.maximum(pd_ref[:, 0:1], 1.0)


def _make_tc_layer0():
  grid = (NPAD // BM,)

  def body(p_ref, pd_ref, x_ref, wl_ref, bl_ref, wr_ref, o_ref):
    inv = _deg_inv(pd_ref)
    x = jnp.concatenate([x_ref[cc] for cc in range(2)], axis=-1)
    acc = jnp.dot(x, wr_ref[...], preferred_element_type=jnp.float32)
    acc += bl_ref[...]
    agg = jnp.concatenate([p_ref[cc] for cc in range(2)], axis=-1) * inv
    acc += jnp.dot(agg, wl_ref[...], preferred_element_type=jnp.float32)
    h = jnp.maximum(acc, 0.0)
    for co in range(4):
      o_ref[co] = h[:, co * 128:(co + 1) * 128]

  return pl.pallas_call(
      body,
      grid=grid,
      in_specs=[
          pl.BlockSpec((2, BM, 128), lambda i: (0, i, 0)),
          pl.BlockSpec((BM, 128), lambda i: (i, 0)),
          pl.BlockSpec((2, BM, 128), lambda i: (0, i, 0)),
          pl.BlockSpec((256, 512), lambda i: (0, 0)),
          pl.BlockSpec((1, 512), lambda i: (0, 0)),
          pl.BlockSpec((256, 512), lambda i: (0, 0)),
      ],
      out_specs=pl.BlockSpec((4, BM, 128), lambda i: (0, i, 0)),
      out_shape=jax.ShapeDtypeStruct((4, NPAD, 128), jnp.float32),
  )


def _make_tc_layer1():
  grid = (NPAD // BM,)

  def body(p_ref, pd_ref, x_ref, wl_ref, bl_ref, wr_ref, w2_ref,
           z_ref, r_ref):
    inv = _deg_inv(pd_ref)
    x = jnp.concatenate([x_ref[cc] for cc in range(4)], axis=-1)
    acc = jnp.dot(x, wr_ref[...], preferred_element_type=jnp.float32)
    acc += bl_ref[...]
    agg = jnp.concatenate([p_ref[cc] for cc in range(4)], axis=-1) * inv
    acc += jnp.dot(agg, wl_ref[...], preferred_element_type=jnp.float32)
    h = jnp.maximum(acc, 0.0)
    zr = jnp.dot(h, w2_ref[...], preferred_element_type=jnp.float32)
    z_ref[...] = zr[:, :128]
    r_ref[...] = zr[:, 128:]

  return pl.pallas_call(
      body,
      grid=grid,
      in_specs=[
          pl.BlockSpec((4, BM, 128), lambda i: (0, i, 0)),
          pl.BlockSpec((BM, 128), lambda i: (i, 0)),
          pl.BlockSpec((4, BM, 128), lambda i: (0, i, 0)),
          pl.BlockSpec((512, 512), lambda i: (0, 0)),
          pl.BlockSpec((1, 512), lambda i: (0, 0)),
          pl.BlockSpec((512, 512), lambda i: (0, 0)),
          pl.BlockSpec((512, 256), lambda i: (0, 0)),
      ],
      out_specs=[
          pl.BlockSpec((BM, 128), lambda i: (i, 0)),
          pl.BlockSpec((BM, 128), lambda i: (i, 0)),
      ],
      out_shape=[
          jax.ShapeDtypeStruct((NPAD, 128), jnp.float32),
          jax.ShapeDtypeStruct((NPAD, 128), jnp.float32),
      ],
  )


def _make_tc_post2():
  grid = (NPAD // BM,)

  def body(p_ref, pd_ref, r_ref, bl_ref, o_ref):
    inv = _deg_inv(pd_ref)
    o_ref[...] = p_ref[...] * inv + r_ref[...] + bl_ref[...]

  return pl.pallas_call(
      body,
      grid=grid,
      in_specs=[
          pl.BlockSpec((BM, 128), lambda i: (i, 0)),
          pl.BlockSpec((BM, 128), lambda i: (i, 0)),
          pl.BlockSpec((BM, 128), lambda i: (i, 0)),
          pl.BlockSpec((1, 128), lambda i: (0, 0)),
      ],
      out_specs=pl.BlockSpec((BM, 128), lambda i: (i, 0)),
      out_shape=jax.ShapeDtypeStruct((NPAD, 128), jnp.float32),
  )


def _chunked(a):
  npad, d = a.shape
  return a.reshape(npad, d // 128, 128).transpose(1, 0, 2)


@jax.jit
def kernel(x, edge_index, Wl0, bl0, Wr0, Wl1, bl1, Wr1, Wl2, bl2, Wr2):
  src = jnp.concatenate(
      [edge_index[0], jnp.zeros((EPAD - N_EDGES,), jnp.int32)])
  dst = jnp.concatenate(
      [edge_index[1], jnp.full((EPAD - N_EDGES,), DUMMY, jnp.int32)])
  packed = (src | (dst << 16)).reshape(NTILE, NB, BEDGE)
  zeros128 = jnp.zeros((128, 128), jnp.float32)
  ones128 = jnp.ones((128, 128), jnp.float32)

  xc = _chunked(jnp.pad(x, ((0, NPAD - N_NODES), (0, 0))))

  p0a, p0b, pdeg = _make_sc_agg(2, True)(xc[0], xc[1], packed,
                                         zeros128, ones128)
  p0 = jnp.stack([p0a, p0b], axis=0)
  h1 = _make_tc_layer0()(p0, pdeg, xc, Wl0, bl0.reshape(1, -1), Wr0)

  p1s = _make_sc_agg(4, False)(h1[0], h1[1], h1[2], h1[3], packed, zeros128)
  p1 = jnp.stack(p1s, axis=0)
  w2 = jnp.concatenate([Wl2, Wr2], axis=1)
  z, r = _make_tc_layer1()(p1, pdeg, h1, Wl1, bl1.reshape(1, -1), Wr1, w2)

  (p2,) = _make_sc_agg(1, False)(z, packed, zeros128)
  out = _make_tc_post2()(p2, pdeg, r, bl2.reshape(1, -1))
  return out[:N_NODES]
